# Initial kernel scaffold; baseline (speedup 1.0000x reference)
import functools

import jax
import jax.numpy as jnp
from jax import lax
from jax.experimental import pallas as pl
from jax.experimental.pallas import tpu as pltpu
from jax.experimental.pallas import tpu_sc as plsc

N = 10000
E = 640000
H = 128
G = 64
L = 5
NCLS = 24


def _ctabs_body(type_ref, dir_ref, w_ref, b_ref, out_ref):
    for i in range(L):
        w = w_ref[i]
        t = jnp.dot(type_ref[:], w, preferred_element_type=jnp.float32)
        d = jnp.dot(dir_ref[:], w, preferred_element_type=jnp.float32)
        out_ref[i] = t[:, None, :] + d[None, :, :] + b_ref[i][None, None, :]


def _ctabs(type_emb, dir_emb, linW, linb):
    return pl.pallas_call(
        _ctabs_body,
        out_shape=jax.ShapeDtypeStruct((L, 6, 4, H), jnp.float32),
    )(type_emb, dir_emb, linW, linb)


_BN_EMB = 1000


def _embed_body(x0_ref, x1_ref, atom_ref, chiral_ref, ctab_ref, h_ref, m_ref):
    a = jnp.clip(x0_ref[:], 0, 119)
    oh_a = (a == lax.broadcasted_iota(jnp.int32, (_BN_EMB, 120), 1)).astype(
        jnp.float32)
    h = jnp.dot(oh_a, atom_ref[:], preferred_element_type=jnp.float32)
    b = jnp.clip(x1_ref[:], 0, 3)
    oh_b = (b == lax.broadcasted_iota(jnp.int32, (_BN_EMB, 4), 1)).astype(
        jnp.float32)
    h = h + jnp.dot(oh_b, chiral_ref[:], preferred_element_type=jnp.float32)
    h_ref[:] = h
    m_ref[:] = jnp.maximum(h[:, None, :] + ctab_ref[:][None], 0.0)


def _embed(x0, x1, atom_emb, chiral_emb, ctab0):
    grid = N // _BN_EMB
    return pl.pallas_call(
        _embed_body,
        grid=(grid,),
        in_specs=[
            pl.BlockSpec((_BN_EMB, 1), lambda i: (i, 0)),
            pl.BlockSpec((_BN_EMB, 1), lambda i: (i, 0)),
            pl.BlockSpec((120, H), lambda i: (0, 0)),
            pl.BlockSpec((4, H), lambda i: (0, 0)),
            pl.BlockSpec((NCLS, H), lambda i: (0, 0)),
        ],
        out_specs=[
            pl.BlockSpec((_BN_EMB, H), lambda i: (i, 0)),
            pl.BlockSpec((_BN_EMB, NCLS, H), lambda i: (i, 0, 0)),
        ],
        out_shape=[
            jax.ShapeDtypeStruct((N, H), jnp.float32),
            jax.ShapeDtypeStruct((N, NCLS, H), jnp.float32),
        ],
    )(x0, x1, atom_emb, chiral_emb, ctab0)


_NGRP = E // 128
_NW = 32
_STEPS = (_NGRP + _NW - 1) // _NW
_RPS = N // 16


def _sc_aggregate(m2, comb2, dst2, zeros_nh):
    mesh = plsc.VectorSubcoreMesh(core_axis_name="c", subcore_axis_name="s")

    @functools.partial(
        pl.kernel,
        out_type=jax.ShapeDtypeStruct((2, N, H), jnp.float32),
        mesh=mesh,
        scratch_types=[
            pltpu.VMEM((128,), jnp.int32),
            pltpu.VMEM((128,), jnp.int32),
            pltpu.VMEM((128, H), jnp.float32),
            pltpu.VMEM_SHARED((N, H), jnp.float32),
            pltpu.SemaphoreType.DMA,
        ],
    )
    def k(m_hbm, comb_hbm, dst_hbm, zero_hbm, out_hbm,
          comb_v, dst_v, rows_v, aggr_sh, sem):
        c = lax.axis_index("c")
        s = lax.axis_index("s")
        w = s * 2 + c

        pltpu.sync_copy(zero_hbm.at[pl.ds(s * _RPS, _RPS)],
                        aggr_sh.at[pl.ds(s * _RPS, _RPS)])
        plsc.subcore_barrier()

        def body(i, carry):
            g = w + i * _NW

            @pl.when(g < _NGRP)
            def _():
                pltpu.sync_copy(comb_hbm.at[g], comb_v)
                pltpu.sync_copy(dst_hbm.at[g], dst_v)
                pltpu.async_copy(m_hbm.at[comb_v], rows_v, sem).wait()
                pltpu.sync_copy(rows_v, aggr_sh.at[dst_v], add=True)

            return carry

        lax.fori_loop(0, _STEPS, body, 0)
        plsc.subcore_barrier()

        pltpu.sync_copy(aggr_sh.at[pl.ds(s * _RPS, _RPS)],
                        out_hbm.at[c, pl.ds(s * _RPS, _RPS)])

    return k(m2, comb2, dst2, zeros_nh)


_BN_MLP = 2000


def _mlp_stats_body(a0_ref, a1_ref, h_ref, w1_ref, b1_ref, w2_ref, b2_ref,
                    out_ref, stats_ref, acc_ref):
    i = pl.program_id(0)
    t = a0_ref[:] + a1_ref[:] + h_ref[:]
    z = jnp.maximum(
        jnp.dot(t, w1_ref[:], preferred_element_type=jnp.float32)
        + b1_ref[:], 0.0)
    o = jnp.dot(z, w2_ref[:], preferred_element_type=jnp.float32) + b2_ref[:]
    out_ref[:] = o
    snew = jnp.concatenate(
        [jnp.sum(o, axis=0, keepdims=True),
         jnp.sum(o * o, axis=0, keepdims=True)], axis=0)

    @pl.when(i == 0)
    def _():
        acc_ref[:] = snew

    @pl.when(i > 0)
    def _():
        acc_ref[:] = acc_ref[:] + snew

    @pl.when(i == (N // _BN_MLP) - 1)
    def _():
        stats_ref[:] = acc_ref[:]


def _mlp_stats(a0, a1, h, w1, b1, w2, b2):
    grid = N // _BN_MLP
    return pl.pallas_call(
        _mlp_stats_body,
        grid=(grid,),
        in_specs=[
            pl.BlockSpec((_BN_MLP, H), lambda i: (i, 0)),
            pl.BlockSpec((_BN_MLP, H), lambda i: (i, 0)),
            pl.BlockSpec((_BN_MLP, H), lambda i: (i, 0)),
            pl.BlockSpec((H, H), lambda i: (0, 0)),
            pl.BlockSpec((1, H), lambda i: (0, 0)),
            pl.BlockSpec((H, H), lambda i: (0, 0)),
            pl.BlockSpec((1, H), lambda i: (0, 0)),
        ],
        out_specs=[
            pl.BlockSpec((_BN_MLP, H), lambda i: (i, 0)),
            pl.BlockSpec((2, H), lambda i: (0, 0)),
        ],
        out_shape=[
            jax.ShapeDtypeStruct((N, H), jnp.float32),
            jax.ShapeDtypeStruct((2, H), jnp.float32),
        ],
        scratch_shapes=[pltpu.VMEM((2, H), jnp.float32)],
    )(a0, a1, h, w1, b1, w2, b2)


_BN_BN = 1000


def _bn_resid_body_m(out_ref, h_ref, stats_ref, g_ref, be_ref, ctab_ref,
                     hn_ref, m_ref):
    mu = stats_ref[0:1, :] / N
    ex2 = stats_ref[1:2, :] / N
    var = ex2 - mu * mu
    rstd = lax.rsqrt(var + 1e-5)
    o = (out_ref[:] - mu) * rstd * g_ref[:] + be_ref[:]
    hn = jnp.maximum(o, 0.0) + h_ref[:]
    hn_ref[:] = hn
    m_ref[:] = jnp.maximum(hn[:, None, :] + ctab_ref[:][None], 0.0)


def _bn_resid_body(out_ref, h_ref, stats_ref, g_ref, be_ref, hn_ref):
    mu = stats_ref[0:1, :] / N
    ex2 = stats_ref[1:2, :] / N
    var = ex2 - mu * mu
    rstd = lax.rsqrt(var + 1e-5)
    o = (out_ref[:] - mu) * rstd * g_ref[:] + be_ref[:]
    hn_ref[:] = jnp.maximum(o, 0.0) + h_ref[:]


def _bn_resid(out, h, stats, gamma, beta, ctab_next):
    grid = N // _BN_BN
    base_specs = [
        pl.BlockSpec((_BN_BN, H), lambda i: (i, 0)),
        pl.BlockSpec((_BN_BN, H), lambda i: (i, 0)),
        pl.BlockSpec((2, H), lambda i: (0, 0)),
        pl.BlockSpec((1, H), lambda i: (0, 0)),
        pl.BlockSpec((1, H), lambda i: (0, 0)),
    ]
    if ctab_next is not None:
        res = pl.pallas_call(
            _bn_resid_body_m,
            grid=(grid,),
            in_specs=base_specs + [pl.BlockSpec((NCLS, H), lambda i: (0, 0))],
            out_specs=[
                pl.BlockSpec((_BN_BN, H), lambda i: (i, 0)),
                pl.BlockSpec((_BN_BN, NCLS, H), lambda i: (i, 0, 0)),
            ],
            out_shape=[
                jax.ShapeDtypeStruct((N, H), jnp.float32),
                jax.ShapeDtypeStruct((N, NCLS, H), jnp.float32),
            ],
        )(out, h, stats, gamma, beta, ctab_next)
        return res[0], res[1]
    hn = pl.pallas_call(
        _bn_resid_body,
        grid=(grid,),
        in_specs=base_specs,
        out_specs=pl.BlockSpec((_BN_BN, H), lambda i: (i, 0)),
        out_shape=jax.ShapeDtypeStruct((N, H), jnp.float32),
    )(out, h, stats, gamma, beta)
    return hn, None


def _pool_head_body(h_ref, batch_ref, exp_ref,
                    ew1_ref, eb1_ref, ew2_ref, eb2_ref,
                    hw1_ref, hb1_ref, hw2_ref, hb2_ref, hw3_ref, hb3_ref,
                    o_ref, gemb_ref, comb_ref,
                    sum_s, max_s, cnt_s):
    hh = h_ref[:]
    b = batch_ref[:]

    def body(g, carry):
        mask = b == g
        msum = jnp.sum(jnp.where(mask, hh, 0.0), axis=0, keepdims=True)
        mmax = jnp.max(jnp.where(mask, hh, -3.4e38), axis=0, keepdims=True)
        cnt = jnp.sum(mask.astype(jnp.float32))
        sum_s[pl.ds(g, 1), :] = msum
        max_s[pl.ds(g, 1), :] = mmax
        cnt_s[pl.ds(g, 1), :] = jnp.full((1, H), cnt, jnp.float32)
        return carry

    lax.fori_loop(0, G, body, 0)
    counts = cnt_s[:]
    hmean = sum_s[:] / jnp.maximum(counts, 1.0)
    hmax = jnp.where(counts > 0.0, max_s[:], 0.0)
    gemb = jnp.concatenate([hmean, hmax], axis=1)
    gemb_ref[:] = gemb
    ex = jnp.maximum(
        jnp.dot(exp_ref[:], ew1_ref[:], preferred_element_type=jnp.float32)
        + eb1_ref[:], 0.0)
    ex = jnp.maximum(
        jnp.dot(ex, ew2_ref[:], preferred_element_type=jnp.float32)
        + eb2_ref[:], 0.0)
    comb = jnp.concatenate([gemb, ex], axis=1)
    comb_ref[:] = comb
    o = jnp.maximum(
        jnp.dot(comb, hw1_ref[:], preferred_element_type=jnp.float32)
        + hb1_ref[:], 0.0)
    o = jnp.maximum(
        jnp.dot(o, hw2_ref[:], preferred_element_type=jnp.float32)
        + hb2_ref[:], 0.0)
    o_ref[:] = jnp.dot(o, hw3_ref[:], preferred_element_type=jnp.float32) \
        + hb3_ref[:]


def _pool_head(h, batch2, exp, ew1, eb1, ew2, eb2,
               hw1, hb1, hw2, hb2, hw3, hb3):
    return pl.pallas_call(
        _pool_head_body,
        out_shape=[
            jax.ShapeDtypeStruct((G, 1), jnp.float32),
            jax.ShapeDtypeStruct((G, 2 * H), jnp.float32),
            jax.ShapeDtypeStruct((G, 2 * H + 256), jnp.float32),
        ],
        scratch_shapes=[
            pltpu.VMEM((G, H), jnp.float32),
            pltpu.VMEM((G, H), jnp.float32),
            pltpu.VMEM((G, H), jnp.float32),
        ],
    )(h, batch2, exp, ew1, eb1, ew2, eb2, hw1, hb1, hw2, hb2, hw3, hb3)


@jax.jit
def kernel(x, edge_index, edge_attr, batch, experimental_feat, atom_emb,
           chiral_emb, edge_type_emb, edge_dir_emb, convW1, convb1, convW2,
           convb2, linW, linb, bn_gamma, bn_beta, expW1, expb1, expW2, expb2,
           headW1, headb1, headW2, headb2, headW3, headb3):
    src = edge_index[0]
    dst = edge_index[1]
    cls = (jnp.clip(edge_attr[:, 0], 0, 5) * 4
           + jnp.clip(edge_attr[:, 1], 0, 3))
    comb2 = (src * NCLS + cls).reshape(_NGRP, 128)
    dst2 = dst.reshape(_NGRP, 128)
    x0 = x[:, 0:1]
    x1 = x[:, 1:2]
    batch2 = batch[:, None]
    zeros_nh = jnp.zeros((N, H), jnp.float32)

    ctabs = _ctabs(edge_type_emb, edge_dir_emb, linW, linb).reshape(
        L, NCLS, H)
    h, m3 = _embed(x0, x1, atom_emb, chiral_emb, ctabs[0])
    m2 = m3.reshape(N * NCLS, H)

    for i in range(L):
        ap = _sc_aggregate(m2, comb2, dst2, zeros_nh)
        out, stats = _mlp_stats(ap[0], ap[1], h, convW1[i],
                                convb1[i][None, :], convW2[i],
                                convb2[i][None, :])
        ctab_next = ctabs[i + 1] if i + 1 < L else None
        h, m3 = _bn_resid(out, h, stats, bn_gamma[i][None, :],
                          bn_beta[i][None, :], ctab_next)
        if m3 is not None:
            m2 = m3.reshape(N * NCLS, H)

    return _pool_head(
        h, batch2, experimental_feat,
        expW1, expb1[None, :], expW2, expb2[None, :],
        headW1, headb1[None, :], headW2, headb2[None, :],
        headW3, headb3[None, :])


# speedup vs baseline: 10.6419x; 10.6419x over previous
import functools

import jax
import jax.numpy as jnp
from jax import lax
from jax.experimental import pallas as pl
from jax.experimental.pallas import tpu as pltpu
from jax.experimental.pallas import tpu_sc as plsc

N = 10000
E = 640000
H = 128
G = 64
L = 5
NCLS = 24


_BN_M = 1000


def _mtab_body(h_ref, ctab_ref, m_ref):
    m_ref[:] = jnp.maximum(h_ref[:][:, None, :] + ctab_ref[:][None], 0.0)


def _mtab(h, ctab):
    grid = N // _BN_M
    return pl.pallas_call(
        _mtab_body,
        grid=(grid,),
        in_specs=[
            pl.BlockSpec((_BN_M, H), lambda i: (i, 0)),
            pl.BlockSpec((NCLS, H), lambda i: (0, 0)),
        ],
        out_specs=pl.BlockSpec((_BN_M, NCLS, H), lambda i: (i, 0, 0)),
        out_shape=jax.ShapeDtypeStruct((N, NCLS, H), jnp.float32),
    )(h, ctab)


_NGRP = E // 128
_NSB = _NGRP // 8
_NW = 32
_STEPS = (_NSB + _NW - 1) // _NW
_RPS = 624


def _sc_aggregate(m2, comb2, dst2, zeros_nh):
    mesh = plsc.VectorSubcoreMesh(core_axis_name="c", subcore_axis_name="s")

    @functools.partial(
        pl.kernel,
        out_type=jax.ShapeDtypeStruct((2, N, H), jnp.float32),
        mesh=mesh,
        scratch_types=[
            pltpu.VMEM((8, 128), jnp.int32),
            pltpu.VMEM((8, 128), jnp.int32),
            pltpu.VMEM((128, H), jnp.float32),
            pltpu.VMEM_SHARED((N, H), jnp.float32),
            pltpu.SemaphoreType.DMA,
        ],
    )
    def k(m_hbm, comb_hbm, dst_hbm, zero_hbm, out_hbm,
          comb_v, dst_v, rows_v, aggr_sh, sem):
        c = lax.axis_index("c")
        s = lax.axis_index("s")
        w = s * 2 + c

        @pl.when(s < 15)
        def _():
            pltpu.sync_copy(zero_hbm.at[pl.ds(s * _RPS, _RPS)],
                            aggr_sh.at[pl.ds(s * _RPS, _RPS)])

        @pl.when(s == 15)
        def _():
            pltpu.sync_copy(zero_hbm.at[pl.ds(15 * _RPS, N - 15 * _RPS)],
                            aggr_sh.at[pl.ds(15 * _RPS, N - 15 * _RPS)])

        plsc.subcore_barrier()

        def body(i, carry):
            sb = w + i * _NW

            @pl.when(sb < _NSB)
            def _():
                pltpu.sync_copy(comb_hbm.at[pl.ds(sb * 8, 8)], comb_v)
                pltpu.sync_copy(dst_hbm.at[pl.ds(sb * 8, 8)], dst_v)
                for j in range(8):
                    pltpu.async_copy(m_hbm.at[comb_v.at[j]], rows_v,
                                     sem).wait()
                    pltpu.sync_copy(rows_v, aggr_sh.at[dst_v.at[j]],
                                    add=True)

            return carry

        lax.fori_loop(0, _STEPS, body, 0)
        plsc.subcore_barrier()

        @pl.when(s < 15)
        def _():
            pltpu.sync_copy(aggr_sh.at[pl.ds(s * _RPS, _RPS)],
                            out_hbm.at[c, pl.ds(s * _RPS, _RPS)])

        @pl.when(s == 15)
        def _():
            pltpu.sync_copy(aggr_sh.at[pl.ds(15 * _RPS, N - 15 * _RPS)],
                            out_hbm.at[c, pl.ds(15 * _RPS, N - 15 * _RPS)])

    return k(m2, comb2, dst2, zeros_nh)


def _pool_head_body(h_ref, batch_ref, exp_ref,
                    ew1_ref, eb1_ref, ew2_ref, eb2_ref,
                    hw1_ref, hb1_ref, hw2_ref, hb2_ref, hw3_ref, hb3_ref,
                    o_ref, gemb_ref, comb_ref,
                    sum_s, max_s, cnt_s):
    hh = h_ref[:]
    b = batch_ref[:]

    def body(g, carry):
        mask = b == g
        msum = jnp.sum(jnp.where(mask, hh, 0.0), axis=0, keepdims=True)
        mmax = jnp.max(jnp.where(mask, hh, -3.4e38), axis=0, keepdims=True)
        cnt = jnp.sum(mask.astype(jnp.float32))
        sum_s[pl.ds(g, 1), :] = msum
        max_s[pl.ds(g, 1), :] = mmax
        cnt_s[pl.ds(g, 1), :] = jnp.full((1, H), cnt, jnp.float32)
        return carry

    lax.fori_loop(0, G, body, 0)
    counts = cnt_s[:]
    hmean = sum_s[:] / jnp.maximum(counts, 1.0)
    hmax = jnp.where(counts > 0.0, max_s[:], 0.0)
    gemb = jnp.concatenate([hmean, hmax], axis=1)
    gemb_ref[:] = gemb
    ex = jnp.maximum(
        jnp.dot(exp_ref[:], ew1_ref[:], preferred_element_type=jnp.float32,
                precision=lax.Precision.HIGHEST)
        + eb1_ref[:], 0.0)
    ex = jnp.maximum(
        jnp.dot(ex, ew2_ref[:], preferred_element_type=jnp.float32,
                precision=lax.Precision.HIGHEST)
        + eb2_ref[:], 0.0)
    comb = jnp.concatenate([gemb, ex], axis=1)
    comb_ref[:] = comb
    o = jnp.maximum(
        jnp.dot(comb, hw1_ref[:], preferred_element_type=jnp.float32,
                precision=lax.Precision.HIGHEST)
        + hb1_ref[:], 0.0)
    o = jnp.maximum(
        jnp.dot(o, hw2_ref[:], preferred_element_type=jnp.float32,
                precision=lax.Precision.HIGHEST)
        + hb2_ref[:], 0.0)
    o_ref[:] = jnp.dot(o, hw3_ref[:], preferred_element_type=jnp.float32,
                       precision=lax.Precision.HIGHEST) + hb3_ref[:]


def _pool_head(h, batch2, exp, ew1, eb1, ew2, eb2,
               hw1, hb1, hw2, hb2, hw3, hb3):
    return pl.pallas_call(
        _pool_head_body,
        out_shape=[
            jax.ShapeDtypeStruct((G, 1), jnp.float32),
            jax.ShapeDtypeStruct((G, 2 * H), jnp.float32),
            jax.ShapeDtypeStruct((G, 2 * H + 256), jnp.float32),
        ],
        scratch_shapes=[
            pltpu.VMEM((G, H), jnp.float32),
            pltpu.VMEM((G, H), jnp.float32),
            pltpu.VMEM((G, H), jnp.float32),
        ],
    )(h, batch2, exp, ew1, eb1, ew2, eb2, hw1, hb1, hw2, hb2, hw3, hb3)


@jax.jit
def kernel(x, edge_index, edge_attr, batch, experimental_feat, atom_emb,
           chiral_emb, edge_type_emb, edge_dir_emb, convW1, convb1, convW2,
           convb2, linW, linb, bn_gamma, bn_beta, expW1, expb1, expW2, expb2,
           headW1, headb1, headW2, headb2, headW3, headb3):
    src = edge_index[0]
    dst = edge_index[1]
    cls = (jnp.clip(edge_attr[:, 0], 0, 5) * 4
           + jnp.clip(edge_attr[:, 1], 0, 3))
    comb2 = (src * NCLS + cls).reshape(_NGRP, 128)
    dst2 = dst.reshape(_NGRP, 128)
    batch2 = batch[:, None]
    zeros_nh = jnp.zeros((N, H), jnp.float32)

    e24 = (edge_type_emb[:, None, :] + edge_dir_emb[None, :, :]).reshape(
        NCLS, H)
    ctabs = jnp.einsum("ch,lhk->lck", e24, linW) + linb[:, None, :]

    h = (atom_emb[jnp.clip(x[:, 0], 0, 119)]
         + chiral_emb[jnp.clip(x[:, 1], 0, 3)])

    for i in range(L):
        m2 = _mtab(h, ctabs[i]).reshape(N * NCLS, H)
        ap = _sc_aggregate(m2, comb2, dst2, zeros_nh)
        out = ap[0] + ap[1] + h
        out = jnp.maximum(out @ convW1[i] + convb1[i], 0.0) \
            @ convW2[i] + convb2[i]
        mu = jnp.mean(out, axis=0)
        var = jnp.var(out, axis=0)
        out = (out - mu) / jnp.sqrt(var + 1e-5) * bn_gamma[i] + bn_beta[i]
        h = jnp.maximum(out, 0.0) + h

    return _pool_head(
        h, batch2, experimental_feat,
        expW1, expb1[None, :], expW2, expb2[None, :],
        headW1, headb1[None, :], headW2, headb2[None, :],
        headW3, headb3[None, :])
